# baseline (speedup 1.0000x reference)
import functools

import jax
import jax.numpy as jnp
from jax import lax
from jax.experimental import pallas as pl
from jax.experimental.pallas import tpu as pltpu
from jax.experimental.pallas import tpu_sc as plsc

N = 8192
B = 16
K = 20
TM = 128
TMR = 512
TN = 640
TP = 32
NEG = -3.0e38
INF = 3.0e38
BIGI = 2**30


def _tc_knn(dp):
    nbt = TN // 128

    def body(c0s, nts, xf, xr_blk, batchf, b2d, idx_ref):
        i = pl.program_id(0)
        c0 = c0s[i]
        nt = nts[i]
        x_r = xr_blk[...]
        br = b2d[...]
        sq_r = jnp.sum(x_r * x_r, axis=1, keepdims=True)

        def col_tile(t, carry):
            bestv, besti = carry
            cs_u = c0 + t * TN
            cs = jnp.maximum(jnp.minimum(cs_u, N - TN), 0)
            x_c = xf[pl.ds(cs, TN), :]
            sq_c = lax.transpose(
                jnp.sum(x_c * x_c, axis=1, keepdims=True), (1, 0))
            G = lax.dot_general(x_r, x_c, (((1,), (1,)), ((), ())),
                                preferred_element_type=jnp.float32)
            E = (sq_r + sq_c) - 2.0 * G
            cb = cs // 128
            bc = jnp.concatenate(
                [batchf[pl.ds(cb + j, 1), :] for j in range(nbt)], axis=1)
            colg = cs + lax.broadcasted_iota(jnp.int32, (1, TN), 1)
            valid = (bc == br) & (colg >= cs_u)
            E = jnp.where(valid, E, INF)
            buf = jnp.concatenate([E, bestv], axis=1)
            ibuf = jnp.concatenate(
                [jnp.broadcast_to(colg, (TMR, TN)), besti], axis=1)
            vs, js = [], []
            for _ in range(K):
                m = jnp.min(buf, axis=1, keepdims=True)
                j = jnp.min(jnp.where(buf == m, ibuf, BIGI), axis=1,
                            keepdims=True)
                vs.append(m)
                js.append(j)
                buf = jnp.where((buf == m) & (ibuf == j), INF, buf)
            padv = jnp.full((TMR, 128 - K), INF, jnp.float32)
            padj = jnp.full((TMR, 128 - K), BIGI, jnp.int32)
            return (jnp.concatenate(vs + [padv], axis=1),
                    jnp.concatenate(js + [padj], axis=1))

        init = (jnp.full((TMR, 128), INF, jnp.float32),
                jnp.full((TMR, 128), BIGI, jnp.int32))
        _, besti = lax.fori_loop(0, nt, col_tile, init)
        idx_ref[...] = jnp.clip(besti[:, :K], 0, N - 1)

    full = lambda shape: pl.BlockSpec(shape, lambda i, *_: (0,) * len(shape))
    grid_spec = pltpu.PrefetchScalarGridSpec(
        num_scalar_prefetch=2,
        grid=(N // TMR,),
        in_specs=[
            full((N, dp)),
            pl.BlockSpec((TMR, dp), lambda i, *_: (i, 0)),
            full((N // TM, TM)),
            pl.BlockSpec((TMR, 1), lambda i, *_: (i, 0)),
        ],
        out_specs=[pl.BlockSpec((TMR, K), lambda i, *_: (i, 0))],
    )
    return pl.pallas_call(
        body,
        grid_spec=grid_spec,
        out_shape=[jax.ShapeDtypeStruct((N, K), jnp.int32)],
        compiler_params=pltpu.CompilerParams(
            dimension_semantics=("parallel",)),
    )


def _sc_edge(dp, dF, P=8):
    mesh = plsc.VectorSubcoreMesh(core_axis_name="c", subcore_axis_name="s")
    NW = 32
    PPW = N // NW
    NWIN = PPW // P
    nch = dF // 16

    @functools.partial(
        pl.kernel,
        mesh=mesh,
        out_type=jax.ShapeDtypeStruct((N * K, dF), jnp.float32),
        scratch_types=[
            pltpu.VMEM((P * K,), jnp.int32),
            pltpu.VMEM((P * K,), jnp.int32),
            pltpu.VMEM((P * K, dp), jnp.float32),
            pltpu.VMEM((P * K, dp), jnp.float32),
            pltpu.VMEM((P, dp), jnp.float32),
            pltpu.VMEM((P * K, dF), jnp.float32),
            pltpu.SemaphoreType.DMA,
            pltpu.SemaphoreType.DMA,
        ],
    )
    def kern(x_hbm, idx_hbm, out_hbm, i0, i1, r0, r1, xi_v, o_v, s0, s1):
        wid = lax.axis_index("s") * 2 + lax.axis_index("c")
        base0 = wid * PPW
        idxs, rows, sems = (i0, i1), (r0, r1), (s0, s1)

        def start_gather(w, b):
            base = base0 + w * P
            pltpu.sync_copy(idx_hbm.at[pl.ds(base * K, P * K)], idxs[b])
            pltpu.async_copy(x_hbm.at[idxs[b]], rows[b], sems[b])

        start_gather(0, 0)

        @pl.loop(0, NWIN, step=2)
        def _(w0):
            for b in range(2):
                w = w0 + b
                ob = 1 - b
                pltpu.make_async_copy(
                    x_hbm.at[idxs[b]], rows[b], sems[b]).wait()

                @pl.when(w + 1 < NWIN)
                def _():
                    start_gather(w + 1, ob)

                base = base0 + w * P
                pltpu.sync_copy(x_hbm.at[pl.ds(base, P), :], xi_v)

                @pl.loop(0, P)
                def _(p):
                    @pl.loop(0, nch)
                    def _(c):
                        sl = pl.ds(c * 16, 16)
                        xi = xi_v[p, sl]
                        for k in range(K):
                            o_v[p * K + k, sl] = rows[b][p * K + k, sl] - xi

                pltpu.sync_copy(o_v, out_hbm.at[pl.ds(base * K, P * K), :])

    return kern


def _tc_edge(dF, dout, dpout):
    TE = TP * K

    def body(x_blk, f_blk, Wa, Wb, bv, out_ref):
        ie = lax.broadcasted_iota(jnp.int32, (TE, TP), 0) // K
        ip = lax.broadcasted_iota(jnp.int32, (TE, TP), 1)
        R = (ie == ip).astype(jnp.float32)
        xi_rep = lax.dot_general(R, x_blk[...], (((1,), (0,)), ((), ())),
                                 preferred_element_type=jnp.float32)
        hA = lax.dot_general(xi_rep, Wa[...], (((1,), (0,)), ((), ())),
                             preferred_element_type=jnp.float32)
        hB = lax.dot_general(f_blk[...], Wb[...], (((1,), (0,)), ((), ())),
                             preferred_element_type=jnp.float32)
        h = hA + hB + bv[...]
        lh = jnp.where(h >= 0, h, 0.02 * h)
        rows = [jnp.max(lh[p * K:(p + 1) * K, :], axis=0, keepdims=True)
                for p in range(TP)]
        xo = jnp.concatenate(rows, axis=0)
        if dpout > dout:
            xo = jnp.concatenate(
                [xo, jnp.zeros((TP, dpout - dout), jnp.float32)], axis=1)
        out_ref[...] = xo

    return pl.pallas_call(
        body,
        grid=(N // TP,),
        in_specs=[
            pl.BlockSpec((TP, dF), lambda i: (i, 0)),
            pl.BlockSpec((TE, dF), lambda i: (i, 0)),
            pl.BlockSpec((dF, dout), lambda i: (0, 0)),
            pl.BlockSpec((dF, dout), lambda i: (0, 0)),
            pl.BlockSpec((1, dout), lambda i: (0, 0)),
        ],
        out_specs=pl.BlockSpec((TP, dpout), lambda i: (i, 0)),
        out_shape=jax.ShapeDtypeStruct((N, dpout), jnp.float32),
        compiler_params=pltpu.CompilerParams(
            dimension_semantics=("parallel",)),
    )


def _pool_head():
    def body(x1, x2, x3, x4, b2d, Wl, blv, gv, bev, out_ref):
        def chunk(c, pooled):
            r0 = c * TM
            rows = jnp.concatenate(
                [x1[pl.ds(r0, TM), :][:, :64], x2[pl.ds(r0, TM), :][:, :64],
                 x3[pl.ds(r0, TM), :], x4[pl.ds(r0, TM), :]], axis=1)
            bb = b2d[pl.ds(r0, TM), :]
            stack = jnp.concatenate(
                [jnp.max(jnp.where(bb == float(b), rows, NEG), axis=0,
                         keepdims=True) for b in range(B)], axis=0)
            return jnp.maximum(pooled, stack)

        pooled = lax.fori_loop(0, N // TM, chunk,
                               jnp.full((B, 512), NEG, jnp.float32))
        y = lax.dot_general(pooled, Wl[...], (((1,), (0,)), ((), ())),
                            preferred_element_type=jnp.float32) + blv[...]
        mean = jnp.mean(y, axis=0, keepdims=True)
        var = jnp.mean((y - mean) ** 2, axis=0, keepdims=True)
        yn = (y - mean) / jnp.sqrt(var + 1e-5) * gv[...] + bev[...]
        out_ref[...] = jnp.maximum(yn, 0.0)

    full = lambda shape: pl.BlockSpec(shape, lambda *_: (0,) * len(shape))
    return pl.pallas_call(
        body,
        grid=(1,),
        in_specs=[full((N, 128)), full((N, 128)), full((N, 128)),
                  full((N, 256)), full((N, 1)), full((512, 256)),
                  full((1, 256)), full((1, 256)), full((1, 256))],
        out_specs=full((B, 256)),
        out_shape=jax.ShapeDtypeStruct((B, 256), jnp.float32),
    )


def _tile_ranges(batch):
    seg_start = jnp.searchsorted(batch, jnp.arange(B, dtype=batch.dtype))
    seg_end = jnp.searchsorted(batch, jnp.arange(B, dtype=batch.dtype),
                               side="right")
    first = batch[:: TMR]
    last = batch[TMR - 1:: TMR]
    c0 = (seg_start[first] // 128) * 128
    c1 = seg_end[last]
    nts = jnp.maximum((c1 - c0 + TN - 1) // TN, 1)
    return c0.astype(jnp.int32), nts.astype(jnp.int32)


def kernel(pos, batch, W1, b1, W2, b2, W3, b3, W4, b4, Wl, bl, gamma, beta):
    batch = batch.astype(jnp.int32)
    c0s, nts = _tile_ranges(batch)
    batchf = batch.astype(jnp.float32).reshape(N // TM, TM)
    b2d = batch.astype(jnp.float32).reshape(N, 1)

    x = jnp.pad(pos, ((0, 0), (0, 125)))
    outs = []
    for (W, bvec, d) in ((W1, b1, 3), (W2, b2, 64), (W3, b3, 64),
                         (W4, b4, 128)):
        dout = W.shape[1]
        dp = x.shape[1]
        dF = max(((d + 15) // 16) * 16, 16)
        dpout = max(dout, 128)
        Wa = jnp.pad(W[:d], ((0, dF - d), (0, 0)))
        Wb = jnp.pad(W[d:], ((0, dF - d), (0, 0)))
        (idx,) = _tc_knn(dp)(c0s, nts, x, x, batchf, b2d)
        F = _sc_edge(dp, dF)(x, idx.reshape(N * K))
        x = _tc_edge(dF, dout, dpout)(x[:, :dF], F, Wa, Wb,
                                      bvec.reshape(1, dout))
        outs.append(x)

    return _pool_head()(outs[0], outs[1], outs[2], outs[3], b2d, Wl,
                        bl.reshape(1, 256), gamma.reshape(1, 256),
                        beta.reshape(1, 256))


# speedup vs baseline: 1.0431x; 1.0431x over previous
import functools

import jax
import jax.numpy as jnp
from jax import lax
from jax.experimental import pallas as pl
from jax.experimental.pallas import tpu as pltpu
from jax.experimental.pallas import tpu_sc as plsc

N = 8192
B = 16
K = 20
TM = 128
TMR = 256
TN = 640
TP = 32
NEG = -3.0e38
INF = 3.0e38
BIGI = 2**30


def _tc_knn(dp):
    nbt = TN // 128

    def body(c0s, nts, xf, xr_blk, batchf, b2d, idx_ref):
        i = pl.program_id(0)
        c0 = c0s[i]
        nt = nts[i]
        x_r = xr_blk[...]
        br = b2d[...]
        sq_r = jnp.sum(x_r * x_r, axis=1, keepdims=True)

        def col_tile(t, carry):
            bestv, besti = carry
            cs_u = c0 + t * TN
            cs = jnp.maximum(jnp.minimum(cs_u, N - TN), 0)
            x_c = xf[pl.ds(cs, TN), :]
            sq_c = lax.transpose(
                jnp.sum(x_c * x_c, axis=1, keepdims=True), (1, 0))
            G = lax.dot_general(x_r, x_c, (((1,), (1,)), ((), ())),
                                preferred_element_type=jnp.float32)
            E = (sq_r + sq_c) - 2.0 * G
            cb = cs // 128
            bc = jnp.concatenate(
                [batchf[pl.ds(cb + j, 1), :] for j in range(nbt)], axis=1)
            colg = cs + lax.broadcasted_iota(jnp.int32, (1, TN), 1)
            valid = (bc == br) & (colg >= cs_u)
            E = jnp.where(valid, E, INF)
            buf = jnp.concatenate([E, bestv], axis=1)
            ibuf = jnp.concatenate(
                [jnp.broadcast_to(colg, (TMR, TN)), besti], axis=1)
            vs, js = [], []
            for _ in range(K):
                m = jnp.min(buf, axis=1, keepdims=True)
                j = jnp.min(jnp.where(buf == m, ibuf, BIGI), axis=1,
                            keepdims=True)
                vs.append(m)
                js.append(j)
                buf = jnp.where((buf == m) & (ibuf == j), INF, buf)
            padv = jnp.full((TMR, 128 - K), INF, jnp.float32)
            padj = jnp.full((TMR, 128 - K), BIGI, jnp.int32)
            return (jnp.concatenate(vs + [padv], axis=1),
                    jnp.concatenate(js + [padj], axis=1))

        init = (jnp.full((TMR, 128), INF, jnp.float32),
                jnp.full((TMR, 128), BIGI, jnp.int32))
        _, besti = lax.fori_loop(0, nt, col_tile, init)
        idx_ref[...] = jnp.clip(besti[:, :K], 0, N - 1)

    full = lambda shape: pl.BlockSpec(shape, lambda i, *_: (0,) * len(shape))
    grid_spec = pltpu.PrefetchScalarGridSpec(
        num_scalar_prefetch=2,
        grid=(N // TMR,),
        in_specs=[
            full((N, dp)),
            pl.BlockSpec((TMR, dp), lambda i, *_: (i, 0)),
            full((N // TM, TM)),
            pl.BlockSpec((TMR, 1), lambda i, *_: (i, 0)),
        ],
        out_specs=[pl.BlockSpec((TMR, K), lambda i, *_: (i, 0))],
    )
    return pl.pallas_call(
        body,
        grid_spec=grid_spec,
        out_shape=[jax.ShapeDtypeStruct((N, K), jnp.int32)],
        compiler_params=pltpu.CompilerParams(
            dimension_semantics=("parallel",)),
    )


def _sc_edge(dp, dF, P=16):
    mesh = plsc.VectorSubcoreMesh(core_axis_name="c", subcore_axis_name="s")
    NW = 32
    PPW = N // NW
    NWIN = PPW // P
    nch = dF // 16

    @functools.partial(
        pl.kernel,
        mesh=mesh,
        out_type=jax.ShapeDtypeStruct((N * K, dF), jnp.float32),
        scratch_types=[
            pltpu.VMEM((P * K,), jnp.int32),
            pltpu.VMEM((P * K,), jnp.int32),
            pltpu.VMEM((P * K, dp), jnp.float32),
            pltpu.VMEM((P * K, dp), jnp.float32),
            pltpu.VMEM((P, dp), jnp.float32),
            pltpu.VMEM((P * K, dF), jnp.float32),
            pltpu.SemaphoreType.DMA,
            pltpu.SemaphoreType.DMA,
        ],
    )
    def kern(x_hbm, idx_hbm, out_hbm, i0, i1, r0, r1, xi_v, o_v, s0, s1):
        wid = lax.axis_index("s") * 2 + lax.axis_index("c")
        base0 = wid * PPW
        idxs, rows, sems = (i0, i1), (r0, r1), (s0, s1)

        def start_gather(w, b):
            base = base0 + w * P
            pltpu.sync_copy(idx_hbm.at[pl.ds(base * K, P * K)], idxs[b])
            pltpu.async_copy(x_hbm.at[idxs[b]], rows[b], sems[b])

        start_gather(0, 0)

        @pl.loop(0, NWIN, step=2)
        def _(w0):
            for b in range(2):
                w = w0 + b
                ob = 1 - b
                pltpu.make_async_copy(
                    x_hbm.at[idxs[b]], rows[b], sems[b]).wait()

                @pl.when(w + 1 < NWIN)
                def _():
                    start_gather(w + 1, ob)

                base = base0 + w * P
                pltpu.sync_copy(x_hbm.at[pl.ds(base, P), :], xi_v)

                @pl.loop(0, P)
                def _(p):
                    @pl.loop(0, nch)
                    def _(c):
                        sl = pl.ds(c * 16, 16)
                        xi = xi_v[p, sl]
                        for k in range(K):
                            o_v[p * K + k, sl] = rows[b][p * K + k, sl] - xi

                pltpu.sync_copy(o_v, out_hbm.at[pl.ds(base * K, P * K), :])

    return kern


def _tc_edge(dF, dout, dpout):
    TE = TP * K

    def body(x_blk, f_blk, Wa, Wb, bv, out_ref):
        ie = lax.broadcasted_iota(jnp.int32, (TE, TP), 0) // K
        ip = lax.broadcasted_iota(jnp.int32, (TE, TP), 1)
        R = (ie == ip).astype(jnp.float32)
        xi_rep = lax.dot_general(R, x_blk[...], (((1,), (0,)), ((), ())),
                                 preferred_element_type=jnp.float32)
        hA = lax.dot_general(xi_rep, Wa[...], (((1,), (0,)), ((), ())),
                             preferred_element_type=jnp.float32)
        hB = lax.dot_general(f_blk[...], Wb[...], (((1,), (0,)), ((), ())),
                             preferred_element_type=jnp.float32)
        h = hA + hB + bv[...]
        lh = jnp.where(h >= 0, h, 0.02 * h)
        rows = [jnp.max(lh[p * K:(p + 1) * K, :], axis=0, keepdims=True)
                for p in range(TP)]
        xo = jnp.concatenate(rows, axis=0)
        if dpout > dout:
            xo = jnp.concatenate(
                [xo, jnp.zeros((TP, dpout - dout), jnp.float32)], axis=1)
        out_ref[...] = xo

    return pl.pallas_call(
        body,
        grid=(N // TP,),
        in_specs=[
            pl.BlockSpec((TP, dF), lambda i: (i, 0)),
            pl.BlockSpec((TE, dF), lambda i: (i, 0)),
            pl.BlockSpec((dF, dout), lambda i: (0, 0)),
            pl.BlockSpec((dF, dout), lambda i: (0, 0)),
            pl.BlockSpec((1, dout), lambda i: (0, 0)),
        ],
        out_specs=pl.BlockSpec((TP, dpout), lambda i: (i, 0)),
        out_shape=jax.ShapeDtypeStruct((N, dpout), jnp.float32),
        compiler_params=pltpu.CompilerParams(
            dimension_semantics=("parallel",)),
    )


def _pool_head():
    def body(x1, x2, x3, x4, b2d, Wl, blv, gv, bev, out_ref):
        def chunk(c, pooled):
            r0 = c * TM
            rows = jnp.concatenate(
                [x1[pl.ds(r0, TM), :][:, :64], x2[pl.ds(r0, TM), :][:, :64],
                 x3[pl.ds(r0, TM), :], x4[pl.ds(r0, TM), :]], axis=1)
            bb = b2d[pl.ds(r0, TM), :]
            stack = jnp.concatenate(
                [jnp.max(jnp.where(bb == float(b), rows, NEG), axis=0,
                         keepdims=True) for b in range(B)], axis=0)
            return jnp.maximum(pooled, stack)

        pooled = lax.fori_loop(0, N // TM, chunk,
                               jnp.full((B, 512), NEG, jnp.float32))
        y = lax.dot_general(pooled, Wl[...], (((1,), (0,)), ((), ())),
                            preferred_element_type=jnp.float32) + blv[...]
        mean = jnp.mean(y, axis=0, keepdims=True)
        var = jnp.mean((y - mean) ** 2, axis=0, keepdims=True)
        yn = (y - mean) / jnp.sqrt(var + 1e-5) * gv[...] + bev[...]
        out_ref[...] = jnp.maximum(yn, 0.0)

    full = lambda shape: pl.BlockSpec(shape, lambda *_: (0,) * len(shape))
    return pl.pallas_call(
        body,
        grid=(1,),
        in_specs=[full((N, 128)), full((N, 128)), full((N, 128)),
                  full((N, 256)), full((N, 1)), full((512, 256)),
                  full((1, 256)), full((1, 256)), full((1, 256))],
        out_specs=full((B, 256)),
        out_shape=jax.ShapeDtypeStruct((B, 256), jnp.float32),
    )


def _tile_ranges(batch):
    seg_start = jnp.searchsorted(batch, jnp.arange(B, dtype=batch.dtype))
    seg_end = jnp.searchsorted(batch, jnp.arange(B, dtype=batch.dtype),
                               side="right")
    first = batch[:: TMR]
    last = batch[TMR - 1:: TMR]
    c0 = (seg_start[first] // 128) * 128
    c1 = seg_end[last]
    nts = jnp.maximum((c1 - c0 + TN - 1) // TN, 1)
    return c0.astype(jnp.int32), nts.astype(jnp.int32)


def kernel(pos, batch, W1, b1, W2, b2, W3, b3, W4, b4, Wl, bl, gamma, beta):
    batch = batch.astype(jnp.int32)
    c0s, nts = _tile_ranges(batch)
    batchf = batch.astype(jnp.float32).reshape(N // TM, TM)
    b2d = batch.astype(jnp.float32).reshape(N, 1)

    x = jnp.pad(pos, ((0, 0), (0, 125)))
    outs = []
    for (W, bvec, d) in ((W1, b1, 3), (W2, b2, 64), (W3, b3, 64),
                         (W4, b4, 128)):
        dout = W.shape[1]
        dp = x.shape[1]
        dF = max(((d + 15) // 16) * 16, 16)
        dpout = max(dout, 128)
        Wa = jnp.pad(W[:d], ((0, dF - d), (0, 0)))
        Wb = jnp.pad(W[d:], ((0, dF - d), (0, 0)))
        (idx,) = _tc_knn(dp)(c0s, nts, x, x, batchf, b2d)
        F = _sc_edge(dp, dF)(x, idx.reshape(N * K))
        x = _tc_edge(dF, dout, dpout)(x[:, :dF], F, Wa, Wb,
                                      bvec.reshape(1, dout))
        outs.append(x)

    return _pool_head()(outs[0], outs[1], outs[2], outs[3], b2d, Wl,
                        bl.reshape(1, 256), gamma.reshape(1, 256),
                        beta.reshape(1, 256))
